# baseline (speedup 1.0000x reference)
import jax
import jax.numpy as jnp
from jax.experimental import pallas as pl
from jax.experimental.pallas import tpu as pltpu

D_MODEL = 1024
D_FF = 2048
NUM_EXPERTS = 8
K = 2
BATCH = 2
SEQ = 2048

FF_BLOCK = 1024
NUM_FF_BLOCKS = D_FF // FF_BLOCK

_INV_SQRT2 = 0.7071067811865476


def _routing_kernel(x_ref, gw_ref, idx_ref, w_ref, xb_ref):
    xf = x_ref[...]
    xb_ref[...] = xf.astype(jnp.bfloat16)
    xm = jnp.mean(xf, axis=1)
    logits = jax.lax.dot_general(
        xm, gw_ref[...], (((1,), (1,)), ((), ())),
        preferred_element_type=jnp.float32)
    iota_e = jax.lax.broadcasted_iota(jnp.int32, (BATCH, NUM_EXPERTS), 1)
    neg_inf = jnp.float32(-jnp.inf)

    max1 = jnp.max(logits, axis=1, keepdims=True)
    idx1 = jnp.min(jnp.where(logits == max1, iota_e, NUM_EXPERTS),
                   axis=1, keepdims=True)
    masked = jnp.where(iota_e == idx1, neg_inf, logits)
    max2 = jnp.max(masked, axis=1, keepdims=True)
    idx2 = jnp.min(jnp.where(masked == max2, iota_e, NUM_EXPERTS),
                   axis=1, keepdims=True)

    e2 = jnp.exp(max2 - max1)
    denom = 1.0 + e2
    w1 = 1.0 / denom
    w2 = e2 / denom

    idx_ref[...] = jnp.concatenate([idx1, idx2], axis=1).astype(jnp.int32)
    w_ref[...] = jnp.concatenate([w1, w2], axis=1)


def _ffn_kernel(idx_ref, wp_ref, x_ref, W1_ref, b1_ref, W2_ref, b2_ref,
                out_ref):
    b = pl.program_id(0)
    k = pl.program_id(1)
    f = pl.program_id(2)

    x = x_ref[0]
    h = jax.lax.dot_general(
        x, W1_ref[0].astype(jnp.bfloat16), (((1,), (1,)), ((), ())),
        preferred_element_type=jnp.float32)
    h = h + b1_ref[0, 0, pl.ds(f * FF_BLOCK, FF_BLOCK)][None, :]
    pair_w = wp_ref[b * K + k]
    t = jax.lax.erf(h * jnp.float32(_INV_SQRT2))
    u = (0.5 * pair_w) * h
    g = u + u * t
    contrib = jax.lax.dot_general(
        g.astype(jnp.bfloat16), W2_ref[0].astype(jnp.bfloat16),
        (((1,), (1,)), ((), ())),
        preferred_element_type=jnp.float32)

    @pl.when((k == 0) & (f == 0))
    def _first():
        out_ref[0] = contrib + (pair_w * b2_ref[0, 0])[None, :]

    @pl.when((k != 0) | (f != 0))
    def _rest():
        bias = jnp.where(f == 0, pair_w, 0.0) * b2_ref[0, 0]
        out_ref[0] += contrib + bias[None, :]


@jax.jit
def kernel(x, gate_w, W1, b1, W2, b2):
    idx, wts, xb = pl.pallas_call(
        _routing_kernel,
        out_shape=(
            jax.ShapeDtypeStruct((BATCH, K), jnp.int32),
            jax.ShapeDtypeStruct((BATCH, K), jnp.float32),
            jax.ShapeDtypeStruct((BATCH, SEQ, D_MODEL), jnp.bfloat16),
        ),
    )(x, gate_w)

    idx_flat = idx.reshape(BATCH * K)
    wts_flat = wts.reshape(BATCH * K)

    grid_spec = pltpu.PrefetchScalarGridSpec(
        num_scalar_prefetch=2,
        grid=(BATCH, K, NUM_FF_BLOCKS),
        in_specs=[
            pl.BlockSpec((1, SEQ, D_MODEL), lambda b, k, f, idx, wp: (b, 0, 0)),
            pl.BlockSpec((1, FF_BLOCK, D_MODEL),
                         lambda b, k, f, idx, wp: (idx[b * K + k], f, 0)),
            pl.BlockSpec((1, 1, D_FF),
                         lambda b, k, f, idx, wp: (idx[b * K + k], 0, 0)),
            pl.BlockSpec((1, D_MODEL, FF_BLOCK),
                         lambda b, k, f, idx, wp: (idx[b * K + k], 0, f)),
            pl.BlockSpec((1, 1, D_MODEL),
                         lambda b, k, f, idx, wp: (idx[b * K + k], 0, 0)),
        ],
        out_specs=pl.BlockSpec((1, SEQ, D_MODEL),
                               lambda b, k, f, idx, wp: (b, 0, 0)),
    )
    out = pl.pallas_call(
        _ffn_kernel,
        grid_spec=grid_spec,
        out_shape=jax.ShapeDtypeStruct((BATCH, SEQ, D_MODEL), jnp.float32),
    )(idx_flat, wts_flat, xb, W1,
      b1.reshape(NUM_EXPERTS, 1, D_FF), W2,
      b2.reshape(NUM_EXPERTS, 1, D_MODEL))
    return out


# speedup vs baseline: 1.0428x; 1.0428x over previous
import jax
import jax.numpy as jnp
from jax.experimental import pallas as pl
from jax.experimental.pallas import tpu as pltpu

D_MODEL = 1024
D_FF = 2048
NUM_EXPERTS = 8
K = 2
BATCH = 2
SEQ = 2048

FF_BLOCK = 1024
NUM_FF_BLOCKS = D_FF // FF_BLOCK

_INV_SQRT2 = 0.7071067811865476


def _routing_kernel(x_ref, gw_ref, idx_ref, w_ref, xb_ref):
    xf = x_ref[...]
    xb_ref[...] = xf.astype(jnp.bfloat16)
    xm = jnp.mean(xf, axis=1)
    logits = jax.lax.dot_general(
        xm, gw_ref[...], (((1,), (1,)), ((), ())),
        preferred_element_type=jnp.float32)
    iota_e = jax.lax.broadcasted_iota(jnp.int32, (BATCH, NUM_EXPERTS), 1)
    neg_inf = jnp.float32(-jnp.inf)

    max1 = jnp.max(logits, axis=1, keepdims=True)
    idx1 = jnp.min(jnp.where(logits == max1, iota_e, NUM_EXPERTS),
                   axis=1, keepdims=True)
    masked = jnp.where(iota_e == idx1, neg_inf, logits)
    max2 = jnp.max(masked, axis=1, keepdims=True)
    idx2 = jnp.min(jnp.where(masked == max2, iota_e, NUM_EXPERTS),
                   axis=1, keepdims=True)

    e2 = jnp.exp(max2 - max1)
    denom = 1.0 + e2
    w1 = 1.0 / denom
    w2 = e2 / denom

    idx_ref[...] = jnp.concatenate([idx1, idx2], axis=1).astype(jnp.int32)
    w_ref[...] = jnp.concatenate([w1, w2], axis=1)


def _ffn_kernel(idx_ref, wp_ref, x_ref, W1_ref, b1_ref, W2_ref, b2_ref,
                out_ref):
    b = pl.program_id(0)
    k = pl.program_id(1)
    f = pl.program_id(2)

    x = x_ref[0]
    h = jax.lax.dot_general(
        x, W1_ref[0].astype(jnp.bfloat16), (((1,), (1,)), ((), ())),
        preferred_element_type=jnp.float32)
    h = (h + b1_ref[0, 0, pl.ds(f * FF_BLOCK, FF_BLOCK)][None, :]).astype(
        jnp.bfloat16)
    pair_w = wp_ref[b * K + k]
    t = jax.lax.erf(h * jnp.bfloat16(_INV_SQRT2))
    u = (0.5 * pair_w).astype(jnp.bfloat16) * h
    g = u + u * t
    contrib = jax.lax.dot_general(
        g, W2_ref[0].astype(jnp.bfloat16),
        (((1,), (1,)), ((), ())),
        preferred_element_type=jnp.float32)

    @pl.when((k == 0) & (f == 0))
    def _first():
        out_ref[0] = contrib + (pair_w * b2_ref[0, 0])[None, :]

    @pl.when((k != 0) | (f != 0))
    def _rest():
        bias = jnp.where(f == 0, pair_w, 0.0) * b2_ref[0, 0]
        out_ref[0] += contrib + bias[None, :]


@jax.jit
def kernel(x, gate_w, W1, b1, W2, b2):
    idx, wts, xb = pl.pallas_call(
        _routing_kernel,
        out_shape=(
            jax.ShapeDtypeStruct((BATCH, K), jnp.int32),
            jax.ShapeDtypeStruct((BATCH, K), jnp.float32),
            jax.ShapeDtypeStruct((BATCH, SEQ, D_MODEL), jnp.bfloat16),
        ),
    )(x, gate_w)

    idx_flat = idx.reshape(BATCH * K)
    wts_flat = wts.reshape(BATCH * K)

    grid_spec = pltpu.PrefetchScalarGridSpec(
        num_scalar_prefetch=2,
        grid=(BATCH, K, NUM_FF_BLOCKS),
        in_specs=[
            pl.BlockSpec((1, SEQ, D_MODEL), lambda b, k, f, idx, wp: (b, 0, 0)),
            pl.BlockSpec((1, FF_BLOCK, D_MODEL),
                         lambda b, k, f, idx, wp: (idx[b * K + k], f, 0)),
            pl.BlockSpec((1, 1, D_FF),
                         lambda b, k, f, idx, wp: (idx[b * K + k], 0, 0)),
            pl.BlockSpec((1, D_MODEL, FF_BLOCK),
                         lambda b, k, f, idx, wp: (idx[b * K + k], 0, f)),
            pl.BlockSpec((1, 1, D_MODEL),
                         lambda b, k, f, idx, wp: (idx[b * K + k], 0, 0)),
        ],
        out_specs=pl.BlockSpec((1, SEQ, D_MODEL),
                               lambda b, k, f, idx, wp: (b, 0, 0)),
    )
    out = pl.pallas_call(
        _ffn_kernel,
        grid_spec=grid_spec,
        out_shape=jax.ShapeDtypeStruct((BATCH, SEQ, D_MODEL), jnp.float32),
    )(idx_flat, wts_flat, xb, W1,
      b1.reshape(NUM_EXPERTS, 1, D_FF), W2,
      b2.reshape(NUM_EXPERTS, 1, D_MODEL))
    return out


# speedup vs baseline: 1.0628x; 1.0192x over previous
import jax
import jax.numpy as jnp
from jax.experimental import pallas as pl
from jax.experimental.pallas import tpu as pltpu

D_MODEL = 1024
D_FF = 2048
NUM_EXPERTS = 8
K = 2
BATCH = 2
SEQ = 2048

FF_BLOCK = 1024
NUM_FF_BLOCKS = D_FF // FF_BLOCK

_INV_SQRT2 = 0.7071067811865476


def _routing_kernel(x_ref, gw_ref, idx_ref, w_ref):
    xm = jnp.mean(x_ref[...], axis=1)
    logits = jax.lax.dot_general(
        xm, gw_ref[...], (((1,), (1,)), ((), ())),
        preferred_element_type=jnp.float32)
    iota_e = jax.lax.broadcasted_iota(jnp.int32, (BATCH, NUM_EXPERTS), 1)
    neg_inf = jnp.float32(-jnp.inf)

    max1 = jnp.max(logits, axis=1, keepdims=True)
    idx1 = jnp.min(jnp.where(logits == max1, iota_e, NUM_EXPERTS),
                   axis=1, keepdims=True)
    masked = jnp.where(iota_e == idx1, neg_inf, logits)
    max2 = jnp.max(masked, axis=1, keepdims=True)
    idx2 = jnp.min(jnp.where(masked == max2, iota_e, NUM_EXPERTS),
                   axis=1, keepdims=True)

    e2 = jnp.exp(max2 - max1)
    denom = 1.0 + e2
    w1 = 1.0 / denom
    w2 = e2 / denom

    idx_ref[...] = jnp.concatenate([idx1, idx2], axis=1).astype(jnp.int32)
    w_ref[...] = jnp.concatenate([w1, w2], axis=1)


def _ffn_kernel(idx_ref, wp_ref, x_ref, W1_ref, b1_ref, W2_ref, b2_ref,
                out_ref):
    b = pl.program_id(0)
    k = pl.program_id(1)
    f = pl.program_id(2)

    x = x_ref[0].astype(jnp.bfloat16)
    h = jax.lax.dot_general(
        x, W1_ref[0].astype(jnp.bfloat16), (((1,), (1,)), ((), ())),
        preferred_element_type=jnp.float32)
    h = (h + b1_ref[0, 0, pl.ds(f * FF_BLOCK, FF_BLOCK)][None, :]).astype(
        jnp.bfloat16)
    pair_w = wp_ref[b * K + k]
    t = jax.lax.erf(h * jnp.bfloat16(_INV_SQRT2))
    u = (0.5 * pair_w).astype(jnp.bfloat16) * h
    g = u + u * t
    contrib = jax.lax.dot_general(
        g, W2_ref[0].astype(jnp.bfloat16),
        (((1,), (1,)), ((), ())),
        preferred_element_type=jnp.float32)

    @pl.when((k == 0) & (f == 0))
    def _first():
        out_ref[0] = contrib + (pair_w * b2_ref[0, 0])[None, :]

    @pl.when((k != 0) | (f != 0))
    def _rest():
        bias = jnp.where(f == 0, pair_w, 0.0) * b2_ref[0, 0]
        out_ref[0] += contrib + bias[None, :]


@jax.jit
def kernel(x, gate_w, W1, b1, W2, b2):
    idx, wts = pl.pallas_call(
        _routing_kernel,
        out_shape=(
            jax.ShapeDtypeStruct((BATCH, K), jnp.int32),
            jax.ShapeDtypeStruct((BATCH, K), jnp.float32),
        ),
    )(x, gate_w)

    idx_flat = idx.reshape(BATCH * K)
    wts_flat = wts.reshape(BATCH * K)

    grid_spec = pltpu.PrefetchScalarGridSpec(
        num_scalar_prefetch=2,
        grid=(BATCH, K, NUM_FF_BLOCKS),
        in_specs=[
            pl.BlockSpec((1, SEQ, D_MODEL), lambda b, k, f, idx, wp: (b, 0, 0)),
            pl.BlockSpec((1, FF_BLOCK, D_MODEL),
                         lambda b, k, f, idx, wp: (idx[b * K + k], f, 0)),
            pl.BlockSpec((1, 1, D_FF),
                         lambda b, k, f, idx, wp: (idx[b * K + k], 0, 0)),
            pl.BlockSpec((1, D_MODEL, FF_BLOCK),
                         lambda b, k, f, idx, wp: (idx[b * K + k], 0, f)),
            pl.BlockSpec((1, 1, D_MODEL),
                         lambda b, k, f, idx, wp: (idx[b * K + k], 0, 0)),
        ],
        out_specs=pl.BlockSpec((1, SEQ, D_MODEL),
                               lambda b, k, f, idx, wp: (b, 0, 0)),
    )
    out = pl.pallas_call(
        _ffn_kernel,
        grid_spec=grid_spec,
        out_shape=jax.ShapeDtypeStruct((BATCH, SEQ, D_MODEL), jnp.float32),
    )(idx_flat, wts_flat, x, W1,
      b1.reshape(NUM_EXPERTS, 1, D_FF), W2,
      b2.reshape(NUM_EXPERTS, 1, D_MODEL))
    return out


# speedup vs baseline: 1.0650x; 1.0021x over previous
import jax
import jax.numpy as jnp
from jax.experimental import pallas as pl
from jax.experimental.pallas import tpu as pltpu

D_MODEL = 1024
D_FF = 2048
NUM_EXPERTS = 8
K = 2
BATCH = 2
SEQ = 2048

FF_BLOCK = 1024
NUM_FF_BLOCKS = D_FF // FF_BLOCK
HALF = FF_BLOCK // 2

_INV_SQRT2 = 0.7071067811865476


def _routing_kernel(x_ref, gw_ref, idx_ref, w_ref):
    xm = jnp.mean(x_ref[...], axis=1)
    logits = jax.lax.dot_general(
        xm, gw_ref[...], (((1,), (1,)), ((), ())),
        preferred_element_type=jnp.float32)
    iota_e = jax.lax.broadcasted_iota(jnp.int32, (BATCH, NUM_EXPERTS), 1)
    neg_inf = jnp.float32(-jnp.inf)

    max1 = jnp.max(logits, axis=1, keepdims=True)
    idx1 = jnp.min(jnp.where(logits == max1, iota_e, NUM_EXPERTS),
                   axis=1, keepdims=True)
    masked = jnp.where(iota_e == idx1, neg_inf, logits)
    max2 = jnp.max(masked, axis=1, keepdims=True)
    idx2 = jnp.min(jnp.where(masked == max2, iota_e, NUM_EXPERTS),
                   axis=1, keepdims=True)

    e2 = jnp.exp(max2 - max1)
    denom = 1.0 + e2
    w1 = 1.0 / denom
    w2 = e2 / denom

    idx_ref[...] = jnp.concatenate([idx1, idx2], axis=1).astype(jnp.int32)
    w_ref[...] = jnp.concatenate([w1, w2], axis=1)


def _gelu_scaled(h, b1_row, pair_w):
    h = (h + b1_row[None, :]).astype(jnp.bfloat16)
    t = jax.lax.erf(h * jnp.bfloat16(_INV_SQRT2))
    u = (0.5 * pair_w).astype(jnp.bfloat16) * h
    return u + u * t


def _ffn_kernel(idx_ref, wp_ref, x_ref, W1a_ref, W1b_ref, b1_ref,
                W2a_ref, W2b_ref, b2_ref, out_ref):
    b = pl.program_id(0)
    k = pl.program_id(1)
    f = pl.program_id(2)

    x = x_ref[0].astype(jnp.bfloat16)
    pair_w = wp_ref[b * K + k]
    dn = (((1,), (1,)), ((), ()))

    ha = jax.lax.dot_general(x, W1a_ref[0, 0].astype(jnp.bfloat16), dn,
                             preferred_element_type=jnp.float32)
    hb = jax.lax.dot_general(x, W1b_ref[0, 0].astype(jnp.bfloat16), dn,
                             preferred_element_type=jnp.float32)
    base = f * FF_BLOCK
    ga = _gelu_scaled(ha, b1_ref[0, 0, pl.ds(base, HALF)], pair_w)
    gb = _gelu_scaled(hb, b1_ref[0, 0, pl.ds(base + HALF, HALF)], pair_w)
    contrib = (
        jax.lax.dot_general(ga, W2a_ref[0].astype(jnp.bfloat16), dn,
                            preferred_element_type=jnp.float32)
        + jax.lax.dot_general(gb, W2b_ref[0].astype(jnp.bfloat16), dn,
                              preferred_element_type=jnp.float32))

    @pl.when((k == 0) & (f == 0))
    def _first():
        out_ref[0] = contrib + (pair_w * b2_ref[0, 0])[None, :]

    @pl.when((k != 0) | (f != 0))
    def _rest():
        bias = jnp.where(f == 0, pair_w, 0.0) * b2_ref[0, 0]
        out_ref[0] += contrib + bias[None, :]


@jax.jit
def kernel(x, gate_w, W1, b1, W2, b2):
    idx, wts = pl.pallas_call(
        _routing_kernel,
        out_shape=(
            jax.ShapeDtypeStruct((BATCH, K), jnp.int32),
            jax.ShapeDtypeStruct((BATCH, K), jnp.float32),
        ),
    )(x, gate_w)

    idx_flat = idx.reshape(BATCH * K)
    wts_flat = wts.reshape(BATCH * K)

    W1v = W1.reshape(NUM_EXPERTS, D_FF // HALF, HALF, D_MODEL)
    grid_spec = pltpu.PrefetchScalarGridSpec(
        num_scalar_prefetch=2,
        grid=(BATCH, K, NUM_FF_BLOCKS),
        in_specs=[
            pl.BlockSpec((1, SEQ, D_MODEL), lambda b, k, f, idx, wp: (b, 0, 0)),
            pl.BlockSpec((1, 1, HALF, D_MODEL),
                         lambda b, k, f, idx, wp: (idx[b * K + k], 2 * f, 0, 0)),
            pl.BlockSpec((1, 1, HALF, D_MODEL),
                         lambda b, k, f, idx, wp: (idx[b * K + k], 2 * f + 1, 0, 0)),
            pl.BlockSpec((1, 1, D_FF),
                         lambda b, k, f, idx, wp: (idx[b * K + k], 0, 0)),
            pl.BlockSpec((1, D_MODEL, HALF),
                         lambda b, k, f, idx, wp: (idx[b * K + k], 0, 2 * f)),
            pl.BlockSpec((1, D_MODEL, HALF),
                         lambda b, k, f, idx, wp: (idx[b * K + k], 0, 2 * f + 1)),
            pl.BlockSpec((1, 1, D_MODEL),
                         lambda b, k, f, idx, wp: (idx[b * K + k], 0, 0)),
        ],
        out_specs=pl.BlockSpec((1, SEQ, D_MODEL),
                               lambda b, k, f, idx, wp: (b, 0, 0)),
    )
    out = pl.pallas_call(
        _ffn_kernel,
        grid_spec=grid_spec,
        out_shape=jax.ShapeDtypeStruct((BATCH, SEQ, D_MODEL), jnp.float32),
    )(idx_flat, wts_flat, x, W1v, W1v,
      b1.reshape(NUM_EXPERTS, 1, D_FF), W2, W2,
      b2.reshape(NUM_EXPERTS, 1, D_MODEL))
    return out
